# baseline (speedup 1.0000x reference)
import jax
import jax.numpy as jnp
from jax.experimental import pallas as pl
from jax.experimental.pallas import tpu as pltpu

N = 10000
G = 64
CAT = 6


def _final_body(pooled_ref, cat_ref, wcat_ref, bcat_ref, wlin_ref, blin_ref, out_ref):
    cat_emb = jax.nn.relu(
        jnp.dot(cat_ref[...], wcat_ref[...], preferred_element_type=jnp.float32)
        + bcat_ref[...]
    )
    z = pooled_ref[...] + cat_emb
    out_ref[...] = (
        jnp.dot(z, wlin_ref[...], preferred_element_type=jnp.float32) + blin_ref[...]
    )


def _gat_conv(x, edge_index, W, a_src, a_dst, b):
    n = x.shape[0]
    loop = jnp.arange(n, dtype=edge_index.dtype)
    src = jnp.concatenate([edge_index[0], loop])
    dst = jnp.concatenate([edge_index[1], loop])
    h = x @ W
    e = (h @ a_src)[src] + (h @ a_dst)[dst]
    e = jax.nn.leaky_relu(e, 0.2)
    m = jax.ops.segment_max(e, dst, num_segments=n)
    m = jnp.where(jnp.isfinite(m), m, 0.0)
    ex = jnp.exp(e - m[dst])
    den = jax.ops.segment_sum(ex, dst, num_segments=n)
    alpha = ex / (den[dst] + 1e-16)
    out = jax.ops.segment_sum(alpha[:, None] * h[src], dst, num_segments=n)
    return out + b


def _layer_norm(x, g, b):
    mu = jnp.mean(x, axis=-1, keepdims=True)
    var = jnp.var(x, axis=-1, keepdims=True)
    return (x - mu) / jnp.sqrt(var + 1e-5) * g + b


def kernel(x, edge_index, batch, cat_features, W0, att_src0, att_dst0, bias0, gamma0, beta0, W1, att_src1, att_dst1, bias1, gamma1, beta1, W_cat, b_cat, W_lin, b_lin):
    layers = [
        (W0, att_src0, att_dst0, bias0, gamma0, beta0),
        (W1, att_src1, att_dst1, bias1, gamma1, beta1),
    ]
    h = x
    for (W, a_s, a_d, b, g, be) in layers:
        h = _gat_conv(h, edge_index, W, a_s, a_d, b)
        h = _layer_norm(h, g, be)
        h = jax.nn.relu(h)
    s = jax.ops.segment_sum(h, batch, num_segments=G)
    c = jax.ops.segment_sum(jnp.ones((N,), dtype=h.dtype), batch, num_segments=G)
    pooled = s / jnp.maximum(c, 1.0)[:, None]

    out = pl.pallas_call(
        _final_body,
        out_shape=jax.ShapeDtypeStruct((G, 128), jnp.float32),
    )(pooled, cat_features.reshape(G, CAT), W_cat, b_cat, W_lin, b_lin)
    return out


# speedup vs baseline: 25.2536x; 25.2536x over previous
import functools

import jax
import jax.numpy as jnp
from jax import lax
from jax.experimental import pallas as pl
from jax.experimental.pallas import tpu as pltpu
from jax.experimental.pallas import tpu_sc as plsc

N = 10000
E = 320000
H = 128
G = 64
CAT = 6

NC = 2
NS = 16
NW = NC * NS
EDGES_PER_TILE = E // NW
CHUNK = 80
NCHUNK = EDGES_PER_TILE // CHUNK
ROWS_MAIN = 624
ROWS_TAIL_OFF = NS * ROWS_MAIN
ROWS_TAIL = N - ROWS_TAIL_OFF


def _pre_body(x_ref, w_ref, as_ref, ad_ref, h_ref, a_ref, b_ref, c_ref, exs_ref):
    h = jnp.dot(x_ref[...], w_ref[...], preferred_element_type=jnp.float32)
    h_ref[...] = h
    a = jnp.dot(h, as_ref[...].reshape(H, 1), preferred_element_type=jnp.float32)
    b = jnp.dot(h, ad_ref[...].reshape(H, 1), preferred_element_type=jnp.float32)
    m = jnp.max(a)
    cb = m + b
    c = jnp.where(cb > 0, cb, 0.2 * cb)
    z = a + b
    zl = jnp.where(z > 0, z, 0.2 * z)
    a_ref[...] = a
    b_ref[...] = b
    c_ref[...] = c
    exs_ref[...] = jnp.exp(zl - c)


def _pre_layer(x, W, att_src, att_dst):
    return pl.pallas_call(
        _pre_body,
        out_shape=[
            jax.ShapeDtypeStruct((N, H), jnp.float32),
            jax.ShapeDtypeStruct((N, 1), jnp.float32),
            jax.ShapeDtypeStruct((N, 1), jnp.float32),
            jax.ShapeDtypeStruct((N, 1), jnp.float32),
            jax.ShapeDtypeStruct((N, 1), jnp.float32),
        ],
    )(x, W, att_src, att_dst)


def _post_body(acc0_ref, acc1_ref, dent_ref, h_ref, exs_ref, bias_ref,
               gamma_ref, beta_ref, out_ref):
    exs = exs_ref[...]
    num = acc0_ref[...] + acc1_ref[...] + exs * h_ref[...]
    den = jnp.sum(dent_ref[...], axis=1, keepdims=True) + exs + 1e-16
    y = num / den + bias_ref[...]
    mu = jnp.mean(y, axis=1, keepdims=True)
    yc = y - mu
    var = jnp.mean(yc * yc, axis=1, keepdims=True)
    y = yc * lax.rsqrt(var + 1e-5) * gamma_ref[...] + beta_ref[...]
    out_ref[...] = jnp.maximum(y, 0.0)


def _post_layer(acc0, acc1, den_t, h, exs, bias, gamma, beta):
    return pl.pallas_call(
        _post_body,
        out_shape=jax.ShapeDtypeStruct((N, H), jnp.float32),
    )(acc0, acc1, den_t, h, exs, bias.reshape(1, H), gamma.reshape(1, H),
      beta.reshape(1, H))


def _final_body(h_ref, batch_ref, cat_ref, wcat_ref, bcat_ref, wlin_ref,
                blin_ref, out_ref):
    bi = batch_ref[...]
    oh = (bi == lax.broadcasted_iota(jnp.int32, (N, G), 1)).astype(jnp.float32)
    s = lax.dot_general(oh, h_ref[...], (((0,), (0,)), ((), ())),
                        preferred_element_type=jnp.float32)
    ones = jnp.ones((N, 1), dtype=jnp.float32)
    cnt = lax.dot_general(oh, ones, (((0,), (0,)), ((), ())),
                          preferred_element_type=jnp.float32)
    pooled = s / jnp.maximum(cnt, 1.0)
    cat_emb = jnp.dot(cat_ref[...], wcat_ref[...],
                      preferred_element_type=jnp.float32) + bcat_ref[...]
    cat_emb = jnp.maximum(cat_emb, 0.0)
    z = pooled + cat_emb
    out_ref[...] = jnp.dot(z, wlin_ref[...],
                           preferred_element_type=jnp.float32) + blin_ref[...]


def _final(h, batch, cat_features, W_cat, b_cat, W_lin, b_lin):
    return pl.pallas_call(
        _final_body,
        out_shape=jax.ShapeDtypeStruct((G, H), jnp.float32),
    )(h, batch.reshape(N, 1), cat_features.reshape(G, CAT), W_cat,
      b_cat.reshape(1, H), W_lin, b_lin.reshape(1, H))


def _sc_body(h_hbm, src_hbm, dst_hbm, a_hbm, b_hbm, c_hbm, zeros_hbm,
             acc_out, den_out,
             a_v, b_v, c_v, den_v, src_v, dst_v, w_v, rows_v, acc_sh, sem):
    cid = lax.axis_index("c")
    sid = lax.axis_index("s")
    wid = cid * NS + sid

    pltpu.sync_copy(a_hbm, a_v)
    pltpu.sync_copy(b_hbm, b_v)
    pltpu.sync_copy(c_hbm, c_v)

    def _zero_body(i, carry):
        off = pl.multiple_of(i * 16, 16)
        den_v[pl.ds(off, 16)] = jnp.zeros((16,), jnp.float32)
        return carry
    lax.fori_loop(0, N // 16, _zero_body, 0)
    pltpu.sync_copy(zeros_hbm.at[pl.ds(sid * ROWS_MAIN, ROWS_MAIN)],
                    acc_sh.at[pl.ds(sid * ROWS_MAIN, ROWS_MAIN)])

    @pl.when(sid == 0)
    def _zero_tail():
        pltpu.sync_copy(zeros_hbm.at[pl.ds(ROWS_TAIL_OFF, ROWS_TAIL)],
                        acc_sh.at[pl.ds(ROWS_TAIL_OFF, ROWS_TAIL)])

    plsc.subcore_barrier()

    base = wid * EDGES_PER_TILE

    def _chunk_body(g, carry):
        off = pl.multiple_of(base + g * CHUNK, 8)
        pltpu.sync_copy(src_hbm.at[pl.ds(off, CHUNK)], src_v)
        pltpu.sync_copy(dst_hbm.at[pl.ds(off, CHUNK)], dst_v)
        cp = pltpu.async_copy(h_hbm.at[src_v], rows_v, sem)
        for t in range(CHUNK // 16):
            sv = src_v[pl.ds(t * 16, 16)]
            dv = dst_v[pl.ds(t * 16, 16)]
            ag = plsc.load_gather(a_v, [sv])
            bg = plsc.load_gather(b_v, [dv])
            cg = plsc.load_gather(c_v, [dv])
            z = ag + bg
            e = jnp.where(z > 0, z, 0.2 * z)
            ex = jnp.exp(e - cg)
            w_v[pl.ds(t * 16, 16)] = ex
            plsc.addupdate_scatter(den_v, [dv], ex)
        cp.wait()

        for t in range(CHUNK // 16):
            w16 = w_v[pl.ds(t * 16, 16)]
            for j in range(16):
                w = w16[j]
                k = t * 16 + j
                for d in range(H // 16):
                    rows_v[k, pl.ds(d * 16, 16)] = rows_v[k, pl.ds(d * 16, 16)] * w

        pltpu.sync_copy(rows_v, acc_sh.at[dst_v], add=True)
        return carry

    lax.fori_loop(0, NCHUNK, _chunk_body, 0)
    plsc.subcore_barrier()

    pltpu.sync_copy(den_v, den_out.at[wid])
    pltpu.sync_copy(acc_sh.at[pl.ds(sid * ROWS_MAIN, ROWS_MAIN)],
                    acc_out.at[cid, pl.ds(sid * ROWS_MAIN, ROWS_MAIN)])

    @pl.when(sid == 0)
    def _export_tail():
        pltpu.sync_copy(acc_sh.at[pl.ds(ROWS_TAIL_OFF, ROWS_TAIL)],
                        acc_out.at[cid, pl.ds(ROWS_TAIL_OFF, ROWS_TAIL)])


@functools.partial(
    pl.kernel,
    mesh=plsc.VectorSubcoreMesh(core_axis_name="c", subcore_axis_name="s"),
    compiler_params=pltpu.CompilerParams(needs_layout_passes=False),
    out_type=[
        jax.ShapeDtypeStruct((NC, N, H), jnp.float32),
        jax.ShapeDtypeStruct((NW, N), jnp.float32),
    ],
    scratch_types=[
        pltpu.VMEM((N,), jnp.float32),
        pltpu.VMEM((N,), jnp.float32),
        pltpu.VMEM((N,), jnp.float32),
        pltpu.VMEM((N,), jnp.float32),
        pltpu.VMEM((CHUNK,), jnp.int32),
        pltpu.VMEM((CHUNK,), jnp.int32),
        pltpu.VMEM((CHUNK,), jnp.float32),
        pltpu.VMEM((CHUNK, H), jnp.float32),
        pltpu.VMEM_SHARED((N, H), jnp.float32),
        pltpu.SemaphoreType.DMA,
    ],
)
def _sc_gat(h_hbm, src_hbm, dst_hbm, a_hbm, b_hbm, c_hbm, zeros_hbm,
            acc_out, den_out, *scratch):
    _sc_body(h_hbm, src_hbm, dst_hbm, a_hbm, b_hbm, c_hbm, zeros_hbm,
             acc_out, den_out, *scratch)


def kernel(x, edge_index, batch, cat_features, W0, att_src0, att_dst0, bias0,
           gamma0, beta0, W1, att_src1, att_dst1, bias1, gamma1, beta1,
           W_cat, b_cat, W_lin, b_lin):
    src = edge_index[0]
    dst = edge_index[1]
    zeros = jnp.zeros((N, H), jnp.float32)

    h = x
    for (W, a_s, a_d, bias, gamma, beta) in (
        (W0, att_src0, att_dst0, bias0, gamma0, beta0),
        (W1, att_src1, att_dst1, bias1, gamma1, beta1),
    ):
        hW, a, b, c, exs = _pre_layer(h, W, a_s, a_d)
        acc, den = _sc_gat(hW, src, dst, a.reshape(N), b.reshape(N),
                           c.reshape(N), zeros)
        h = _post_layer(acc[0], acc[1], den.T, hW, exs, bias, gamma, beta)

    return _final(h, batch, cat_features, W_cat, b_cat, W_lin, b_lin)
